# Initial kernel scaffold; baseline (speedup 1.0000x reference)
import jax
import jax.numpy as jnp
from jax.experimental import pallas as pl


def kernel(x):
    raise NotImplementedError("write your pallas kernel here")


# speedup vs baseline: 8.5419x; 8.5419x over previous
import jax
import jax.numpy as jnp
from jax.experimental import pallas as pl
from jax.experimental.pallas import tpu as pltpu

BR = 256
BC = 2048
LANE = 128


def _tri(n, dtype):
    r = jax.lax.broadcasted_iota(jnp.int32, (n, n), 0)
    c = jax.lax.broadcasted_iota(jnp.int32, (n, n), 1)
    return (r <= c).astype(dtype)


def _scan_kernel(x_ref, o_ref, carry_ref):
    j = pl.program_id(1)

    @pl.when(j == 0)
    def _():
        carry_ref[...] = jnp.zeros_like(carry_ref)

    g = BC // LANE
    x3 = x_ref[...].reshape(BR, g, LANE)
    t_lane = _tri(LANE, jnp.float32)
    local = jax.lax.dot_general(
        x3, t_lane, (((2,), (0,)), ((), ())),
        preferred_element_type=jnp.float32,
    )
    chunk = local[:, :, LANE - 1]
    t_g = _tri(g, jnp.float32)
    incl = jax.lax.dot_general(
        chunk, t_g, (((1,), (0,)), ((), ())),
        preferred_element_type=jnp.float32,
    )
    excl = incl - chunk
    carry = carry_ref[:, :1]
    out3 = local + excl[:, :, None] + carry[:, :, None]
    o_ref[...] = out3.reshape(BR, BC)
    carry_ref[...] = carry_ref[...] + incl[:, g - 1 :]


def kernel(x):
    n, m = x.shape
    grid = (n // BR, m // BC)
    return pl.pallas_call(
        _scan_kernel,
        grid=grid,
        in_specs=[pl.BlockSpec((BR, BC), lambda i, j: (i, j))],
        out_specs=pl.BlockSpec((BR, BC), lambda i, j: (i, j)),
        out_shape=jax.ShapeDtypeStruct((n, m), x.dtype),
        scratch_shapes=[pltpu.VMEM((BR, LANE), jnp.float32)],
        compiler_params=pltpu.CompilerParams(
            dimension_semantics=("parallel", "arbitrary"),
        ),
    )(x)


# speedup vs baseline: 9.2656x; 1.0847x over previous
import jax
import jax.numpy as jnp
from jax.experimental import pallas as pl
from jax.experimental.pallas import tpu as pltpu

BR = 256
BC = 2048
LANE = 128


def _tri(n, dtype):
    r = jax.lax.broadcasted_iota(jnp.int32, (n, n), 0)
    c = jax.lax.broadcasted_iota(jnp.int32, (n, n), 1)
    return (r <= c).astype(dtype)


def _scan_kernel(x_ref, o_ref, carry_ref):
    j = pl.program_id(1)

    @pl.when(j == 0)
    def _():
        carry_ref[...] = jnp.zeros_like(carry_ref)

    t = _tri(LANE, jnp.float32)
    run = carry_ref[:, :1]
    for k in range(BC // LANE):
        sl = slice(k * LANE, (k + 1) * LANE)
        y = jax.lax.dot_general(
            x_ref[:, sl], t, (((1,), (0,)), ((), ())),
            preferred_element_type=jnp.float32,
        )
        y = y + run
        o_ref[:, sl] = y
        run = y[:, LANE - 1 :]
    carry_ref[...] = jnp.broadcast_to(run, carry_ref.shape)


def kernel(x):
    n, m = x.shape
    grid = (n // BR, m // BC)
    return pl.pallas_call(
        _scan_kernel,
        grid=grid,
        in_specs=[pl.BlockSpec((BR, BC), lambda i, j: (i, j))],
        out_specs=pl.BlockSpec((BR, BC), lambda i, j: (i, j)),
        out_shape=jax.ShapeDtypeStruct((n, m), x.dtype),
        scratch_shapes=[pltpu.VMEM((BR, LANE), jnp.float32)],
        compiler_params=pltpu.CompilerParams(
            dimension_semantics=("parallel", "arbitrary"),
        ),
    )(x)


# speedup vs baseline: 12.5823x; 1.3580x over previous
import jax
import jax.numpy as jnp
from jax.experimental import pallas as pl
from jax.experimental.pallas import tpu as pltpu

BR = 512
BC = 2048
LANE = 128


def _tri(n, dtype):
    r = jax.lax.broadcasted_iota(jnp.int32, (n, n), 0)
    c = jax.lax.broadcasted_iota(jnp.int32, (n, n), 1)
    return (r <= c).astype(dtype)


def _scan_kernel(x_ref, o_ref, carry_ref):
    j = pl.program_id(1)

    @pl.when(j == 0)
    def _():
        carry_ref[...] = jnp.zeros_like(carry_ref)

    t = _tri(LANE, jnp.float32)
    run = carry_ref[:, :1]
    for k in range(BC // LANE):
        sl = slice(k * LANE, (k + 1) * LANE)
        y = jax.lax.dot_general(
            x_ref[:, sl], t, (((1,), (0,)), ((), ())),
            preferred_element_type=jnp.float32,
        )
        y = y + run
        o_ref[:, sl] = y
        run = y[:, LANE - 1 :]
    carry_ref[...] = jnp.broadcast_to(run, carry_ref.shape)


def kernel(x):
    n, m = x.shape
    grid = (n // BR, m // BC)
    return pl.pallas_call(
        _scan_kernel,
        grid=grid,
        in_specs=[pl.BlockSpec((BR, BC), lambda i, j: (i, j))],
        out_specs=pl.BlockSpec((BR, BC), lambda i, j: (i, j)),
        out_shape=jax.ShapeDtypeStruct((n, m), x.dtype),
        scratch_shapes=[pltpu.VMEM((BR, LANE), jnp.float32)],
        compiler_params=pltpu.CompilerParams(
            dimension_semantics=("parallel", "arbitrary"),
        ),
    )(x)


# speedup vs baseline: 13.2751x; 1.0551x over previous
import jax
import jax.numpy as jnp
from jax.experimental import pallas as pl
from jax.experimental.pallas import tpu as pltpu

BR = 512
BC = 4096
LANE = 128


def _tri(n, dtype):
    r = jax.lax.broadcasted_iota(jnp.int32, (n, n), 0)
    c = jax.lax.broadcasted_iota(jnp.int32, (n, n), 1)
    return (r <= c).astype(dtype)


def _scan_kernel(x_ref, o_ref, carry_ref):
    j = pl.program_id(1)

    @pl.when(j == 0)
    def _():
        carry_ref[...] = jnp.zeros_like(carry_ref)

    t = _tri(LANE, jnp.float32)
    run = carry_ref[:, :1]
    for k in range(BC // LANE):
        sl = slice(k * LANE, (k + 1) * LANE)
        y = jax.lax.dot_general(
            x_ref[:, sl], t, (((1,), (0,)), ((), ())),
            preferred_element_type=jnp.float32,
        )
        y = y + run
        o_ref[:, sl] = y
        run = y[:, LANE - 1 :]
    carry_ref[...] = jnp.broadcast_to(run, carry_ref.shape)


def kernel(x):
    n, m = x.shape
    grid = (n // BR, m // BC)
    return pl.pallas_call(
        _scan_kernel,
        grid=grid,
        in_specs=[pl.BlockSpec((BR, BC), lambda i, j: (i, j))],
        out_specs=pl.BlockSpec((BR, BC), lambda i, j: (i, j)),
        out_shape=jax.ShapeDtypeStruct((n, m), x.dtype),
        scratch_shapes=[pltpu.VMEM((BR, LANE), jnp.float32)],
        compiler_params=pltpu.CompilerParams(
            dimension_semantics=("parallel", "arbitrary"),
        ),
    )(x)
